# Initial kernel scaffold; baseline (speedup 1.0000x reference)
import jax
import jax.numpy as jnp
from jax.experimental import pallas as pl


def kernel(x, edge_index, edge_attr, batch, W_pre, b_pre, W_f0, b_f0, W_s0, b_s0, W_f1, b_f1, W_s1, b_s1, W_f2, b_f2, W_s2, b_s2, W_post, b_post, W_out, b_out):
    raise NotImplementedError("write your pallas kernel here")


# speedup vs baseline: 3.3199x; 3.3199x over previous
import functools

import jax
import jax.numpy as jnp
from jax import lax
from jax.experimental import pallas as pl
from jax.experimental.pallas import tpu as pltpu
from jax.experimental.pallas import tpu_sc as plsc

N = 10000
E = 320000
D = 128
DE = 16
C = 128
POST = 64
G = 64

_SC_INFO = plsc.get_sparse_core_info()
NC = _SC_INFO.num_cores
NS = _SC_INFO.num_subcores
NW = NC * NS
EPW = E // NW
CE = 400
NCHUNK = EPW // CE
CE_S = 200
NCHUNK_S = EPW // CE_S
ROWS_PER_TILE = N // NS

_mesh = plsc.VectorSubcoreMesh(core_axis_name="c", subcore_axis_name="s")


@functools.partial(
    pl.kernel,
    out_type=[
        jax.ShapeDtypeStruct((E, C), jnp.float32),
        jax.ShapeDtypeStruct((E, C), jnp.float32),
    ],
    mesh=_mesh,
    scratch_types=[
        pltpu.VMEM((CE,), jnp.int32),
        pltpu.VMEM((CE,), jnp.int32),
        pltpu.VMEM((CE, C), jnp.float32),
        pltpu.SemaphoreType.DMA,
    ],
)
def _sc_gather(table, dst, src, gd, gs, idx_d, idx_s, rows, sem):
    cid = lax.axis_index("c")
    sid = lax.axis_index("s")
    wid = sid * NC + cid
    base0 = wid * EPW

    def body(j, _):
        base = base0 + j * CE
        pltpu.sync_copy(dst.at[pl.ds(base, CE)], idx_d)
        pltpu.sync_copy(src.at[pl.ds(base, CE)], idx_s)
        pltpu.async_copy(table.at[idx_d], rows, sem).wait()
        pltpu.sync_copy(rows, gd.at[pl.ds(base, CE)])
        pltpu.async_copy(table.at[idx_s], rows, sem).wait()
        pltpu.sync_copy(rows, gs.at[pl.ds(base, CE)])
        return 0

    lax.fori_loop(0, NCHUNK, body, 0)


@functools.partial(
    pl.kernel,
    out_type=jax.ShapeDtypeStruct((NC, N, C), jnp.float32),
    mesh=_mesh,
    scratch_types=[
        pltpu.VMEM_SHARED((N, C), jnp.float32),
        pltpu.VMEM((CE_S,), jnp.int32),
        pltpu.VMEM((CE_S, C), jnp.float32),
    ],
)
def _sc_scatter(msg, dst, zeros, out, acc, idx_v, rows):
    cid = lax.axis_index("c")
    sid = lax.axis_index("s")
    wid = sid * NC + cid
    base0 = wid * EPW

    @pl.when(sid == 0)
    def _():
        pltpu.sync_copy(zeros, acc)

    plsc.subcore_barrier()

    def body(j, _):
        base = base0 + j * CE_S
        pltpu.sync_copy(dst.at[pl.ds(base, CE_S)], idx_v)
        pltpu.sync_copy(msg.at[pl.ds(base, CE_S)], rows)
        pltpu.sync_copy(rows, acc.at[idx_v], add=True)
        return 0

    lax.fori_loop(0, NCHUNK_S, body, 0)
    plsc.subcore_barrier()

    def wout(i, _):
        @pl.when(i % NS == sid)
        def _():
            r0 = i * 400
            pltpu.sync_copy(acc.at[pl.ds(r0, 400)],
                            out.at[cid, pl.ds(r0, 400)])
        return 0

    lax.fori_loop(0, N // 400, wout, 0)


def _prefc_body(x_ref, w_ref, b_ref, o_ref):
    o_ref[...] = jax.nn.relu(
        jnp.dot(x_ref[...], w_ref[...], preferred_element_type=jnp.float32)
        + b_ref[...]
    )


def _prefc(x, wT, b):
    bn = 2000
    return pl.pallas_call(
        _prefc_body,
        grid=(N // bn,),
        in_specs=[
            pl.BlockSpec((bn, D), lambda i: (i, 0)),
            pl.BlockSpec((D, C), lambda i: (0, 0)),
            pl.BlockSpec((1, C), lambda i: (0, 0)),
        ],
        out_specs=pl.BlockSpec((bn, C), lambda i: (i, 0)),
        out_shape=jax.ShapeDtypeStruct((N, C), jnp.float32),
    )(x, wT, b)


def _edge_body(gd_ref, gs_ref, ea_ref,
               wfd_ref, wfs_ref, wfe_ref, bf_ref,
               wsd_ref, wss_ref, wse_ref, bs_ref, msg_ref):
    gd = gd_ref[...]
    gs = gs_ref[...]
    ea = ea_ref[...]
    f = (jnp.dot(gd, wfd_ref[...], preferred_element_type=jnp.float32)
         + jnp.dot(gs, wfs_ref[...], preferred_element_type=jnp.float32)
         + jnp.dot(ea, wfe_ref[...], preferred_element_type=jnp.float32)
         + bf_ref[...])
    s = (jnp.dot(gd, wsd_ref[...], preferred_element_type=jnp.float32)
         + jnp.dot(gs, wss_ref[...], preferred_element_type=jnp.float32)
         + jnp.dot(ea, wse_ref[...], preferred_element_type=jnp.float32)
         + bs_ref[...])
    msg_ref[...] = jax.nn.sigmoid(f) * jax.nn.softplus(s)


def _edge(gd, gs, ea, wfd, wfs, wfe, bf, wsd, wss, wse, bs):
    be = 4000
    wspec = lambda shape: pl.BlockSpec(shape, lambda i: (0, 0))
    return pl.pallas_call(
        _edge_body,
        grid=(E // be,),
        in_specs=[
            pl.BlockSpec((be, C), lambda i: (i, 0)),
            pl.BlockSpec((be, C), lambda i: (i, 0)),
            pl.BlockSpec((be, DE), lambda i: (i, 0)),
            wspec((C, C)), wspec((C, C)), wspec((DE, C)), wspec((1, C)),
            wspec((C, C)), wspec((C, C)), wspec((DE, C)), wspec((1, C)),
        ],
        out_specs=pl.BlockSpec((be, C), lambda i: (i, 0)),
        out_shape=jax.ShapeDtypeStruct((E, C), jnp.float32),
    )(gd, gs, ea, wfd, wfs, wfe, bf, wsd, wss, wse, bs)


def _update_body(out_ref, p_ref, o_ref):
    o_ref[...] = jax.nn.relu(out_ref[...] + p_ref[0] + p_ref[1])


def _update(out, partials):
    bn = 2000
    return pl.pallas_call(
        _update_body,
        grid=(N // bn,),
        in_specs=[
            pl.BlockSpec((bn, C), lambda i: (i, 0)),
            pl.BlockSpec((NC, bn, C), lambda i: (0, i, 0)),
        ],
        out_specs=pl.BlockSpec((bn, C), lambda i: (i, 0)),
        out_shape=jax.ShapeDtypeStruct((N, C), jnp.float32),
    )(out, partials)


def _post_body(h_ref, batch_ref, wpT_ref, bp_ref, wo_ref, bo_ref, o_ref):
    h = jax.nn.relu(
        jnp.dot(h_ref[...], wpT_ref[...], preferred_element_type=jnp.float32)
        + bp_ref[...]
    )
    seg = batch_ref[...]
    onehot = (seg == lax.broadcasted_iota(jnp.int32, (N, G), 1)).astype(
        jnp.float32
    )
    seg_sum = jax.lax.dot_general(
        onehot, h, (((0,), (0,)), ((), ())),
        preferred_element_type=jnp.float32)
    counts = jnp.sum(onehot, axis=0)
    pooled = seg_sum / jnp.maximum(counts, 1.0)[:, None]
    res = jnp.sum(pooled * wo_ref[...], axis=1) + bo_ref[0, 0]
    o_ref[...] = res[None, :]


def _post(h, batch2d, wpT, bp, wo, bo):
    return pl.pallas_call(
        _post_body,
        in_specs=[
            pl.BlockSpec((N, C), lambda: (0, 0)),
            pl.BlockSpec((N, 1), lambda: (0, 0)),
            pl.BlockSpec((C, POST), lambda: (0, 0)),
            pl.BlockSpec((1, POST), lambda: (0, 0)),
            pl.BlockSpec((1, POST), lambda: (0, 0)),
            pl.BlockSpec((1, 1), lambda: (0, 0)),
        ],
        out_specs=pl.BlockSpec((1, G), lambda: (0, 0)),
        out_shape=jax.ShapeDtypeStruct((1, G), jnp.float32),
    )(h, batch2d, wpT, bp, wo, bo)


def kernel(x, edge_index, edge_attr, batch,
           W_pre, b_pre,
           W_f0, b_f0, W_s0, b_s0,
           W_f1, b_f1, W_s1, b_s1,
           W_f2, b_f2, W_s2, b_s2,
           W_post, b_post, W_out, b_out):
    src = edge_index[0]
    dst = edge_index[1]
    zeros = jnp.zeros((N, C), jnp.float32)

    out = _prefc(x, W_pre.T, b_pre[None, :])

    for (Wf, bf, Ws, bs) in [(W_f0, b_f0, W_s0, b_s0),
                             (W_f1, b_f1, W_s1, b_s1),
                             (W_f2, b_f2, W_s2, b_s2)]:
        wfd = Wf[:, :C].T
        wfs = Wf[:, C:2 * C].T
        wfe = Wf[:, 2 * C:].T
        wsd = Ws[:, :C].T
        wss = Ws[:, C:2 * C].T
        wse = Ws[:, 2 * C:].T
        gd, gs = _sc_gather(out, dst, src)
        msg = _edge(gd, gs, edge_attr,
                    wfd, wfs, wfe, bf[None, :],
                    wsd, wss, wse, bs[None, :])
        partials = _sc_scatter(msg, dst, zeros)
        out = _update(out, partials)

    res = _post(out, batch[:, None], W_post.T, b_post[None, :],
                W_out, b_out[:, None])
    return res.reshape(-1)


# speedup vs baseline: 3.6816x; 1.1089x over previous
import functools

import jax
import jax.numpy as jnp
from jax import lax
from jax.experimental import pallas as pl
from jax.experimental.pallas import tpu as pltpu
from jax.experimental.pallas import tpu_sc as plsc

N = 10000
E = 320000
D = 128
DE = 16
C = 128
POST = 64
G = 64

_SC_INFO = plsc.get_sparse_core_info()
NC = _SC_INFO.num_cores
NS = _SC_INFO.num_subcores
NW = NC * NS
EPW = E // NW
CE = 400
NCHUNK = EPW // CE
CE_S = 160
NCHUNK_S = E // CE_S
KMAX_S = -(-NCHUNK_S // NW)
ROWS_PER_TILE = N // NS

_mesh = plsc.VectorSubcoreMesh(core_axis_name="c", subcore_axis_name="s")


@functools.partial(
    pl.kernel,
    out_type=[
        jax.ShapeDtypeStruct((E, C), jnp.float32),
        jax.ShapeDtypeStruct((E, C), jnp.float32),
    ],
    mesh=_mesh,
    scratch_types=[
        pltpu.VMEM((CE,), jnp.int32),
        pltpu.VMEM((CE,), jnp.int32),
        pltpu.VMEM((CE, C), jnp.float32),
        pltpu.VMEM((CE, C), jnp.float32),
        pltpu.SemaphoreType.DMA,
        pltpu.SemaphoreType.DMA,
        pltpu.SemaphoreType.DMA,
        pltpu.SemaphoreType.DMA,
    ],
)
def _sc_gather(table, dst, src, gd, gs,
               idx0, idx1, rows0, rows1, gsem0, gsem1, wsem0, wsem1):
    cid = lax.axis_index("c")
    sid = lax.axis_index("s")
    wid = sid * NC + cid
    base0 = wid * EPW
    idxb = (idx0, idx1)
    rowsb = (rows0, rows1)
    gsem = (gsem0, gsem1)
    wsem = (wsem0, wsem1)
    eidx = (dst, src)
    gout = (gd, gs)

    def g_start(t, p):
        base = base0 + t * CE
        pltpu.sync_copy(eidx[p].at[pl.ds(base, CE)], idxb[p])
        pltpu.async_copy(table.at[idxb[p]], rowsb[p], gsem[p])

    def g_wait(p):
        pltpu.make_async_copy(table.at[idxb[p]], rowsb[p], gsem[p]).wait()

    def w_start(t, p):
        pltpu.async_copy(rowsb[p], gout[p].at[pl.ds(base0 + t * CE, CE)],
                         wsem[p])

    def w_wait(t, p):
        pltpu.make_async_copy(rowsb[p],
                              gout[p].at[pl.ds(base0 + t * CE, CE)],
                              wsem[p]).wait()

    g_start(0, 0)

    def body(t, _):
        @pl.when(t >= 1)
        def _():
            w_wait(t - 1, 1)
        g_start(t, 1)
        g_wait(0)
        w_start(t, 0)
        @pl.when(t < NCHUNK - 1)
        def _():
            w_wait(t, 0)
            g_start(t + 1, 0)
        g_wait(1)
        w_start(t, 1)
        return 0

    lax.fori_loop(0, NCHUNK, body, 0)
    w_wait(NCHUNK - 1, 0)
    w_wait(NCHUNK - 1, 1)


@functools.partial(
    pl.kernel,
    out_type=jax.ShapeDtypeStruct((NC, N, C), jnp.float32),
    mesh=_mesh,
    scratch_types=[
        pltpu.VMEM_SHARED((N, C), jnp.float32),
        pltpu.VMEM((CE_S,), jnp.int32),
        pltpu.VMEM((CE_S,), jnp.int32),
        pltpu.VMEM((CE_S, C), jnp.float32),
        pltpu.VMEM((CE_S, C), jnp.float32),
        pltpu.SemaphoreType.DMA,
        pltpu.SemaphoreType.DMA,
    ],
)
def _sc_scatter(msg, dst, zeros, out, acc, idx0, idx1, rows0, rows1,
                msem0, msem1):
    cid = lax.axis_index("c")
    sid = lax.axis_index("s")
    wid = sid * NC + cid
    idxb = (idx0, idx1)
    rowsb = (rows0, rows1)
    msem = (msem0, msem1)

    @pl.when(sid == 0)
    def _():
        pltpu.sync_copy(zeros, acc)

    plsc.subcore_barrier()

    def m_start(k, b):
        base = (wid + NW * k) * CE_S
        pltpu.sync_copy(dst.at[pl.ds(base, CE_S)], idxb[b])
        pltpu.async_copy(msg.at[pl.ds(base, CE_S)], rowsb[b], msem[b])

    def m_wait(k, b):
        base = (wid + NW * k) * CE_S
        pltpu.make_async_copy(msg.at[pl.ds(base, CE_S)], rowsb[b],
                              msem[b]).wait()

    def have(k):
        return wid + NW * k < NCHUNK_S

    m_start(0, 0)

    def substep(k, b):
        @pl.when(have(k))
        def _():
            m_wait(k, b)

        @pl.when(have(k + 1))
        def _():
            m_start(k + 1, 1 - b)

        @pl.when(have(k))
        def _():
            pltpu.sync_copy(rowsb[b], acc.at[idxb[b]], add=True)

    def body(i, _):
        substep(2 * i, 0)
        substep(2 * i + 1, 1)
        return 0

    lax.fori_loop(0, (KMAX_S + 1) // 2, body, 0)
    plsc.subcore_barrier()

    def wout(i, _):
        @pl.when(i % NS == sid)
        def _():
            r0 = i * 400
            pltpu.sync_copy(acc.at[pl.ds(r0, 400)],
                            out.at[cid, pl.ds(r0, 400)])
        return 0

    lax.fori_loop(0, N // 400, wout, 0)


def _prefc_body(x_ref, w_ref, b_ref, o_ref):
    o_ref[...] = jax.nn.relu(
        jnp.dot(x_ref[...], w_ref[...], preferred_element_type=jnp.float32)
        + b_ref[...]
    )


def _prefc(x, wT, b):
    bn = 2000
    return pl.pallas_call(
        _prefc_body,
        grid=(N // bn,),
        in_specs=[
            pl.BlockSpec((bn, D), lambda i: (i, 0)),
            pl.BlockSpec((D, C), lambda i: (0, 0)),
            pl.BlockSpec((1, C), lambda i: (0, 0)),
        ],
        out_specs=pl.BlockSpec((bn, C), lambda i: (i, 0)),
        out_shape=jax.ShapeDtypeStruct((N, C), jnp.float32),
    )(x, wT, b)


def _edge_body(gd_ref, gs_ref, ea_ref,
               wfd_ref, wfs_ref, wfe_ref, bf_ref,
               wsd_ref, wss_ref, wse_ref, bs_ref, msg_ref):
    gd = gd_ref[...]
    gs = gs_ref[...]
    ea = ea_ref[...]
    f = (jnp.dot(gd, wfd_ref[...], preferred_element_type=jnp.float32)
         + jnp.dot(gs, wfs_ref[...], preferred_element_type=jnp.float32)
         + jnp.dot(ea, wfe_ref[...], preferred_element_type=jnp.float32)
         + bf_ref[...])
    s = (jnp.dot(gd, wsd_ref[...], preferred_element_type=jnp.float32)
         + jnp.dot(gs, wss_ref[...], preferred_element_type=jnp.float32)
         + jnp.dot(ea, wse_ref[...], preferred_element_type=jnp.float32)
         + bs_ref[...])
    msg_ref[...] = jax.nn.sigmoid(f) * jax.nn.softplus(s)


def _edge(gd, gs, ea, wfd, wfs, wfe, bf, wsd, wss, wse, bs):
    be = 4000
    wspec = lambda shape: pl.BlockSpec(shape, lambda i: (0, 0))
    return pl.pallas_call(
        _edge_body,
        grid=(E // be,),
        in_specs=[
            pl.BlockSpec((be, C), lambda i: (i, 0)),
            pl.BlockSpec((be, C), lambda i: (i, 0)),
            pl.BlockSpec((be, DE), lambda i: (i, 0)),
            wspec((C, C)), wspec((C, C)), wspec((DE, C)), wspec((1, C)),
            wspec((C, C)), wspec((C, C)), wspec((DE, C)), wspec((1, C)),
        ],
        out_specs=pl.BlockSpec((be, C), lambda i: (i, 0)),
        out_shape=jax.ShapeDtypeStruct((E, C), jnp.float32),
    )(gd, gs, ea, wfd, wfs, wfe, bf, wsd, wss, wse, bs)


def _update_body(out_ref, p_ref, o_ref):
    o_ref[...] = jax.nn.relu(out_ref[...] + p_ref[0] + p_ref[1])


def _update(out, partials):
    bn = 2000
    return pl.pallas_call(
        _update_body,
        grid=(N // bn,),
        in_specs=[
            pl.BlockSpec((bn, C), lambda i: (i, 0)),
            pl.BlockSpec((NC, bn, C), lambda i: (0, i, 0)),
        ],
        out_specs=pl.BlockSpec((bn, C), lambda i: (i, 0)),
        out_shape=jax.ShapeDtypeStruct((N, C), jnp.float32),
    )(out, partials)


def _post_body(h_ref, batch_ref, wpT_ref, bp_ref, wo_ref, bo_ref, o_ref):
    h = jax.nn.relu(
        jnp.dot(h_ref[...], wpT_ref[...], preferred_element_type=jnp.float32)
        + bp_ref[...]
    )
    seg = batch_ref[...]
    onehot = (seg == lax.broadcasted_iota(jnp.int32, (N, G), 1)).astype(
        jnp.float32
    )
    seg_sum = jax.lax.dot_general(
        onehot, h, (((0,), (0,)), ((), ())),
        preferred_element_type=jnp.float32)
    counts = jnp.sum(onehot, axis=0)
    pooled = seg_sum / jnp.maximum(counts, 1.0)[:, None]
    res = jnp.sum(pooled * wo_ref[...], axis=1) + bo_ref[0, 0]
    o_ref[...] = res[None, :]


def _post(h, batch2d, wpT, bp, wo, bo):
    return pl.pallas_call(
        _post_body,
        in_specs=[
            pl.BlockSpec((N, C), lambda: (0, 0)),
            pl.BlockSpec((N, 1), lambda: (0, 0)),
            pl.BlockSpec((C, POST), lambda: (0, 0)),
            pl.BlockSpec((1, POST), lambda: (0, 0)),
            pl.BlockSpec((1, POST), lambda: (0, 0)),
            pl.BlockSpec((1, 1), lambda: (0, 0)),
        ],
        out_specs=pl.BlockSpec((1, G), lambda: (0, 0)),
        out_shape=jax.ShapeDtypeStruct((1, G), jnp.float32),
    )(h, batch2d, wpT, bp, wo, bo)


def kernel(x, edge_index, edge_attr, batch,
           W_pre, b_pre,
           W_f0, b_f0, W_s0, b_s0,
           W_f1, b_f1, W_s1, b_s1,
           W_f2, b_f2, W_s2, b_s2,
           W_post, b_post, W_out, b_out):
    src = edge_index[0]
    dst = edge_index[1]
    zeros = jnp.zeros((N, C), jnp.float32)

    out = _prefc(x, W_pre.T, b_pre[None, :])

    for (Wf, bf, Ws, bs) in [(W_f0, b_f0, W_s0, b_s0),
                             (W_f1, b_f1, W_s1, b_s1),
                             (W_f2, b_f2, W_s2, b_s2)]:
        wfd = Wf[:, :C].T
        wfs = Wf[:, C:2 * C].T
        wfe = Wf[:, 2 * C:].T
        wsd = Ws[:, :C].T
        wss = Ws[:, C:2 * C].T
        wse = Ws[:, 2 * C:].T
        gd, gs = _sc_gather(out, dst, src)
        msg = _edge(gd, gs, edge_attr,
                    wfd, wfs, wfe, bf[None, :],
                    wsd, wss, wse, bs[None, :])
        partials = _sc_scatter(msg, dst, zeros)
        out = _update(out, partials)

    res = _post(out, batch[:, None], W_post.T, b_post[None, :],
                W_out, b_out[:, None])
    return res.reshape(-1)


# speedup vs baseline: 4.0844x; 1.1094x over previous
import functools

import jax
import jax.numpy as jnp
from jax import lax
from jax.experimental import pallas as pl
from jax.experimental.pallas import tpu as pltpu
from jax.experimental.pallas import tpu_sc as plsc

N = 10000
E = 320000
D = 128
DE = 16
C = 128
POST = 64
G = 64

_SC_INFO = plsc.get_sparse_core_info()
NC = _SC_INFO.num_cores
NS = _SC_INFO.num_subcores
NW = NC * NS
NHALF = 2
EH = E // NHALF
CE = 400
NCH_G = EH // CE
KMAX_G = -(-NCH_G // NW)
CE_S = 160
NCH_S = EH // CE_S
KMAX_S = -(-NCH_S // NW)

_mesh = plsc.VectorSubcoreMesh(core_axis_name="c", subcore_axis_name="s")


def _make_gather(eoff):
    @functools.partial(
        pl.kernel,
        out_type=[
            jax.ShapeDtypeStruct((E, C), jnp.float32),
            jax.ShapeDtypeStruct((E, C), jnp.float32),
        ],
        mesh=_mesh,
        scratch_types=[
            pltpu.VMEM((CE,), jnp.int32),
            pltpu.VMEM((CE,), jnp.int32),
            pltpu.VMEM((CE, C), jnp.float32),
            pltpu.VMEM((CE, C), jnp.float32),
            pltpu.SemaphoreType.DMA,
            pltpu.SemaphoreType.DMA,
            pltpu.SemaphoreType.DMA,
            pltpu.SemaphoreType.DMA,
        ],
    )
    def gather(table, dst, src, gd, gs,
               idx0, idx1, rows0, rows1, gsem0, gsem1, wsem0, wsem1):
        cid = lax.axis_index("c")
        sid = lax.axis_index("s")
        wid = sid * NC + cid
        idxb = (idx0, idx1)
        rowsb = (rows0, rows1)
        gsem = (gsem0, gsem1)
        wsem = (wsem0, wsem1)
        eidx = (dst, src)
        gout = (gd, gs)

        def base(k):
            return eoff + (wid + NW * k) * CE

        def have(k):
            return wid + NW * k < NCH_G

        def g_start(k, p):
            pltpu.sync_copy(eidx[p].at[pl.ds(base(k), CE)], idxb[p])
            pltpu.async_copy(table.at[idxb[p]], rowsb[p], gsem[p])

        def g_wait(p):
            pltpu.make_async_copy(table.at[idxb[p]], rowsb[p],
                                  gsem[p]).wait()

        def w_start(k, p):
            pltpu.async_copy(rowsb[p], gout[p].at[pl.ds(base(k), CE)],
                             wsem[p])

        def w_wait(k, p):
            pltpu.make_async_copy(rowsb[p],
                                  gout[p].at[pl.ds(base(k), CE)],
                                  wsem[p]).wait()

        g_start(0, 0)

        def body(k, _):
            @pl.when((k >= 1) & have(k))
            def _():
                w_wait(k - 1, 1)

            @pl.when(have(k))
            def _():
                g_start(k, 1)
                g_wait(0)
                w_start(k, 0)

            @pl.when(have(k + 1))
            def _():
                w_wait(k, 0)
                g_start(k + 1, 0)

            @pl.when(have(k))
            def _():
                g_wait(1)
                w_start(k, 1)
            return 0

        lax.fori_loop(0, KMAX_G, body, 0)
        klast = (NCH_G - 1 - wid) // NW
        w_wait(klast, 0)
        w_wait(klast, 1)

    return gather


def _make_scatter(eoff):
    @functools.partial(
        pl.kernel,
        out_type=jax.ShapeDtypeStruct((NC, N, C), jnp.float32),
        mesh=_mesh,
        scratch_types=[
            pltpu.VMEM_SHARED((N, C), jnp.float32),
            pltpu.VMEM((CE_S,), jnp.int32),
            pltpu.VMEM((CE_S,), jnp.int32),
            pltpu.VMEM((CE_S, C), jnp.float32),
            pltpu.VMEM((CE_S, C), jnp.float32),
            pltpu.SemaphoreType.DMA,
            pltpu.SemaphoreType.DMA,
        ],
    )
    def scatter(msg, dst, zeros, out, acc, idx0, idx1, rows0, rows1,
                msem0, msem1):
        cid = lax.axis_index("c")
        sid = lax.axis_index("s")
        wid = sid * NC + cid
        idxb = (idx0, idx1)
        rowsb = (rows0, rows1)
        msem = (msem0, msem1)

        @pl.when(sid == 0)
        def _():
            pltpu.sync_copy(zeros, acc)

        plsc.subcore_barrier()

        def base(k):
            return eoff + (wid + NW * k) * CE_S

        def have(k):
            return wid + NW * k < NCH_S

        def m_start(k, b):
            pltpu.sync_copy(dst.at[pl.ds(base(k), CE_S)], idxb[b])
            pltpu.async_copy(msg.at[pl.ds(base(k), CE_S)], rowsb[b],
                             msem[b])

        def m_wait(k, b):
            pltpu.make_async_copy(msg.at[pl.ds(base(k), CE_S)], rowsb[b],
                                  msem[b]).wait()

        m_start(0, 0)

        def substep(k, b):
            @pl.when(have(k))
            def _():
                m_wait(k, b)

            @pl.when(have(k + 1))
            def _():
                m_start(k + 1, 1 - b)

            @pl.when(have(k))
            def _():
                pltpu.sync_copy(rowsb[b], acc.at[idxb[b]], add=True)

        def body(i, _):
            substep(2 * i, 0)
            substep(2 * i + 1, 1)
            return 0

        lax.fori_loop(0, (KMAX_S + 1) // 2, body, 0)
        plsc.subcore_barrier()

        def wout(i, _):
            @pl.when(i % NS == sid)
            def _():
                r0 = i * 400
                pltpu.sync_copy(acc.at[pl.ds(r0, 400)],
                                out.at[cid, pl.ds(r0, 400)])
            return 0

        lax.fori_loop(0, N // 400, wout, 0)

    return scatter


_gather_h = tuple(_make_gather(h * EH) for h in range(NHALF))
_scatter_h = tuple(_make_scatter(h * EH) for h in range(NHALF))


def _prefc_body(x_ref, w_ref, b_ref, o_ref):
    o_ref[...] = jax.nn.relu(
        jnp.dot(x_ref[...], w_ref[...], preferred_element_type=jnp.float32)
        + b_ref[...]
    )


def _prefc(x, wT, b):
    bn = 2000
    return pl.pallas_call(
        _prefc_body,
        grid=(N // bn,),
        in_specs=[
            pl.BlockSpec((bn, D), lambda i: (i, 0)),
            pl.BlockSpec((D, C), lambda i: (0, 0)),
            pl.BlockSpec((1, C), lambda i: (0, 0)),
        ],
        out_specs=pl.BlockSpec((bn, C), lambda i: (i, 0)),
        out_shape=jax.ShapeDtypeStruct((N, C), jnp.float32),
    )(x, wT, b)


def _edge_body(gd_ref, gs_ref, ea_ref,
               wfd_ref, wfs_ref, wfe_ref, bf_ref,
               wsd_ref, wss_ref, wse_ref, bs_ref, msg_ref):
    gd = gd_ref[...]
    gs = gs_ref[...]
    ea = ea_ref[...]
    f = (jnp.dot(gd, wfd_ref[...], preferred_element_type=jnp.float32)
         + jnp.dot(gs, wfs_ref[...], preferred_element_type=jnp.float32)
         + jnp.dot(ea, wfe_ref[...], preferred_element_type=jnp.float32)
         + bf_ref[...])
    s = (jnp.dot(gd, wsd_ref[...], preferred_element_type=jnp.float32)
         + jnp.dot(gs, wss_ref[...], preferred_element_type=jnp.float32)
         + jnp.dot(ea, wse_ref[...], preferred_element_type=jnp.float32)
         + bs_ref[...])
    msg_ref[...] = jax.nn.sigmoid(f) * jax.nn.softplus(s)


def _edge(h, gd, gs, ea, wfd, wfs, wfe, bf, wsd, wss, wse, bs):
    be = 4000
    boff = h * (EH // be)
    espec = lambda: pl.BlockSpec((be, C), lambda i: (i + boff, 0))
    wspec = lambda shape: pl.BlockSpec(shape, lambda i: (0, 0))
    return pl.pallas_call(
        _edge_body,
        grid=(EH // be,),
        in_specs=[
            espec(),
            espec(),
            pl.BlockSpec((be, DE), lambda i: (i + boff, 0)),
            wspec((C, C)), wspec((C, C)), wspec((DE, C)), wspec((1, C)),
            wspec((C, C)), wspec((C, C)), wspec((DE, C)), wspec((1, C)),
        ],
        out_specs=pl.BlockSpec((be, C), lambda i: (i + boff, 0)),
        out_shape=jax.ShapeDtypeStruct((E, C), jnp.float32),
    )(gd, gs, ea, wfd, wfs, wfe, bf, wsd, wss, wse, bs)


def _update_body(out_ref, p0_ref, p1_ref, o_ref):
    o_ref[...] = jax.nn.relu(out_ref[...] + p0_ref[0] + p0_ref[1]
                             + p1_ref[0] + p1_ref[1])


def _update(out, part0, part1):
    bn = 2000
    pspec = pl.BlockSpec((NC, bn, C), lambda i: (0, i, 0))
    return pl.pallas_call(
        _update_body,
        grid=(N // bn,),
        in_specs=[
            pl.BlockSpec((bn, C), lambda i: (i, 0)),
            pspec,
            pspec,
        ],
        out_specs=pl.BlockSpec((bn, C), lambda i: (i, 0)),
        out_shape=jax.ShapeDtypeStruct((N, C), jnp.float32),
    )(out, part0, part1)


def _post_body(h_ref, batch_ref, wpT_ref, bp_ref, wo_ref, bo_ref, o_ref):
    h = jax.nn.relu(
        jnp.dot(h_ref[...], wpT_ref[...], preferred_element_type=jnp.float32)
        + bp_ref[...]
    )
    seg = batch_ref[...]
    onehot = (seg == lax.broadcasted_iota(jnp.int32, (N, G), 1)).astype(
        jnp.float32
    )
    seg_sum = jax.lax.dot_general(
        onehot, h, (((0,), (0,)), ((), ())),
        preferred_element_type=jnp.float32)
    counts = jnp.sum(onehot, axis=0)
    pooled = seg_sum / jnp.maximum(counts, 1.0)[:, None]
    res = jnp.sum(pooled * wo_ref[...], axis=1) + bo_ref[0, 0]
    o_ref[...] = res[None, :]


def _post(h, batch2d, wpT, bp, wo, bo):
    return pl.pallas_call(
        _post_body,
        in_specs=[
            pl.BlockSpec((N, C), lambda: (0, 0)),
            pl.BlockSpec((N, 1), lambda: (0, 0)),
            pl.BlockSpec((C, POST), lambda: (0, 0)),
            pl.BlockSpec((1, POST), lambda: (0, 0)),
            pl.BlockSpec((1, POST), lambda: (0, 0)),
            pl.BlockSpec((1, 1), lambda: (0, 0)),
        ],
        out_specs=pl.BlockSpec((1, G), lambda: (0, 0)),
        out_shape=jax.ShapeDtypeStruct((1, G), jnp.float32),
    )(h, batch2d, wpT, bp, wo, bo)


def kernel(x, edge_index, edge_attr, batch,
           W_pre, b_pre,
           W_f0, b_f0, W_s0, b_s0,
           W_f1, b_f1, W_s1, b_s1,
           W_f2, b_f2, W_s2, b_s2,
           W_post, b_post, W_out, b_out):
    src = edge_index[0]
    dst = edge_index[1]
    zeros = jnp.zeros((N, C), jnp.float32)

    out = _prefc(x, W_pre.T, b_pre[None, :])

    for (Wf, bf, Ws, bs) in [(W_f0, b_f0, W_s0, b_s0),
                             (W_f1, b_f1, W_s1, b_s1),
                             (W_f2, b_f2, W_s2, b_s2)]:
        wfd = Wf[:, :C].T
        wfs = Wf[:, C:2 * C].T
        wfe = Wf[:, 2 * C:].T
        wsd = Ws[:, :C].T
        wss = Ws[:, C:2 * C].T
        wse = Ws[:, 2 * C:].T
        parts = []
        gh = [None] * NHALF
        msg = [None] * NHALF
        for h in range(NHALF):
            gh[h] = _gather_h[h](out, dst, src)
        for h in range(NHALF):
            msg[h] = _edge(h, gh[h][0], gh[h][1], edge_attr,
                           wfd, wfs, wfe, bf[None, :],
                           wsd, wss, wse, bs[None, :])
        for h in range(NHALF):
            parts.append(_scatter_h[h](msg[h], dst, zeros))
        out = _update(out, parts[0], parts[1])

    res = _post(out, batch[:, None], W_post.T, b_post[None, :],
                W_out, b_out[:, None])
    return res.reshape(-1)


# speedup vs baseline: 4.1540x; 1.0170x over previous
import functools

import jax
import jax.numpy as jnp
from jax import lax
from jax.experimental import pallas as pl
from jax.experimental.pallas import tpu as pltpu
from jax.experimental.pallas import tpu_sc as plsc

N = 10000
E = 320000
D = 128
DE = 16
C = 128
POST = 64
G = 64

_SC_INFO = plsc.get_sparse_core_info()
NC = _SC_INFO.num_cores
NS = _SC_INFO.num_subcores
NW = NC * NS
NHALF = 2
EH = E // NHALF
CE = 400
NCH_G = EH // CE
KMAX_G = -(-NCH_G // NW)
CE_S = 160
NCH_S = EH // CE_S
KMAX_S = -(-NCH_S // NW)

_mesh = plsc.VectorSubcoreMesh(core_axis_name="c", subcore_axis_name="s")


def _make_gather(eoff):
    @functools.partial(
        pl.kernel,
        out_type=[
            jax.ShapeDtypeStruct((E, C), jnp.float32),
            jax.ShapeDtypeStruct((E, C), jnp.float32),
        ],
        mesh=_mesh,
        scratch_types=[
            pltpu.VMEM((CE,), jnp.int32),
            pltpu.VMEM((CE,), jnp.int32),
            pltpu.VMEM((CE,), jnp.int32),
            pltpu.VMEM((CE,), jnp.int32),
            pltpu.VMEM((CE, C), jnp.float32),
            pltpu.VMEM((CE, C), jnp.float32),
            pltpu.SemaphoreType.DMA,
            pltpu.SemaphoreType.DMA,
            pltpu.SemaphoreType.DMA,
            pltpu.SemaphoreType.DMA,
            pltpu.SemaphoreType.DMA,
            pltpu.SemaphoreType.DMA,
            pltpu.SemaphoreType.DMA,
            pltpu.SemaphoreType.DMA,
        ],
    )
    def gather(table, dst, src, gd, gs,
               idx00, idx01, idx10, idx11, rows0, rows1,
               gsem0, gsem1, wsem0, wsem1,
               isem00, isem01, isem10, isem11):
        cid = lax.axis_index("c")
        sid = lax.axis_index("s")
        wid = sid * NC + cid
        idxb = ((idx00, idx01), (idx10, idx11))
        rowsb = (rows0, rows1)
        gsem = (gsem0, gsem1)
        wsem = (wsem0, wsem1)
        isem = ((isem00, isem01), (isem10, isem11))
        eidx = (dst, src)
        gout = (gd, gs)

        def base(k):
            return eoff + (wid + NW * k) * CE

        def have(k):
            return wid + NW * k < NCH_G

        def i_start(k, p, q):
            pltpu.async_copy(eidx[p].at[pl.ds(base(k), CE)],
                             idxb[p][q], isem[p][q])

        def i_wait(k, p, q):
            pltpu.make_async_copy(eidx[p].at[pl.ds(base(k), CE)],
                                  idxb[p][q], isem[p][q]).wait()

        def g_start(k, p, q):
            i_wait(k, p, q)
            pltpu.async_copy(table.at[idxb[p][q]], rowsb[p], gsem[p])

        def g_wait(p, q):
            pltpu.make_async_copy(table.at[idxb[p][q]], rowsb[p],
                                  gsem[p]).wait()

        def w_start(k, p):
            pltpu.async_copy(rowsb[p], gout[p].at[pl.ds(base(k), CE)],
                             wsem[p])

        def w_wait(k, p):
            pltpu.make_async_copy(rowsb[p],
                                  gout[p].at[pl.ds(base(k), CE)],
                                  wsem[p]).wait()

        i_start(0, 0, 0)
        i_start(0, 1, 0)
        g_start(0, 0, 0)

        def chunk_step(k, q):
            @pl.when((k >= 1) & have(k))
            def _():
                w_wait(k - 1, 1)

            @pl.when(have(k))
            def _():
                g_start(k, 1, q)
                g_wait(0, q)
                w_start(k, 0)

            @pl.when(have(k + 1))
            def _():
                i_start(k + 1, 0, 1 - q)
                w_wait(k, 0)
                g_start(k + 1, 0, 1 - q)

            @pl.when(have(k))
            def _():
                g_wait(1, q)
                w_start(k, 1)

            @pl.when(have(k + 1))
            def _():
                i_start(k + 1, 1, 1 - q)

        def body(i, _):
            chunk_step(2 * i, 0)
            chunk_step(2 * i + 1, 1)
            return 0

        lax.fori_loop(0, (KMAX_G + 1) // 2, body, 0)
        klast = (NCH_G - 1 - wid) // NW
        w_wait(klast, 0)
        w_wait(klast, 1)

    return gather


def _make_scatter(eoff):
    @functools.partial(
        pl.kernel,
        out_type=jax.ShapeDtypeStruct((NC, N, C), jnp.float32),
        mesh=_mesh,
        scratch_types=[
            pltpu.VMEM_SHARED((N, C), jnp.float32),
            pltpu.VMEM((CE_S,), jnp.int32),
            pltpu.VMEM((CE_S,), jnp.int32),
            pltpu.VMEM((CE_S, C), jnp.float32),
            pltpu.VMEM((CE_S, C), jnp.float32),
            pltpu.SemaphoreType.DMA,
            pltpu.SemaphoreType.DMA,
            pltpu.SemaphoreType.DMA,
            pltpu.SemaphoreType.DMA,
        ],
    )
    def scatter(msg, dst, zeros, out, acc, idx0, idx1, rows0, rows1,
                msem0, msem1, isem0, isem1):
        cid = lax.axis_index("c")
        sid = lax.axis_index("s")
        wid = sid * NC + cid
        idxb = (idx0, idx1)
        rowsb = (rows0, rows1)
        msem = (msem0, msem1)
        isem = (isem0, isem1)

        @pl.when(sid == 0)
        def _():
            pltpu.sync_copy(zeros, acc)

        plsc.subcore_barrier()

        def base(k):
            return eoff + (wid + NW * k) * CE_S

        def have(k):
            return wid + NW * k < NCH_S

        def m_start(k, b):
            pltpu.async_copy(dst.at[pl.ds(base(k), CE_S)], idxb[b],
                             isem[b])
            pltpu.async_copy(msg.at[pl.ds(base(k), CE_S)], rowsb[b],
                             msem[b])

        def m_wait(k, b):
            pltpu.make_async_copy(dst.at[pl.ds(base(k), CE_S)], idxb[b],
                                  isem[b]).wait()
            pltpu.make_async_copy(msg.at[pl.ds(base(k), CE_S)], rowsb[b],
                                  msem[b]).wait()

        m_start(0, 0)

        def substep(k, b):
            @pl.when(have(k + 1))
            def _():
                m_start(k + 1, 1 - b)

            @pl.when(have(k))
            def _():
                m_wait(k, b)
                pltpu.sync_copy(rowsb[b], acc.at[idxb[b]], add=True)

        def body(i, _):
            substep(2 * i, 0)
            substep(2 * i + 1, 1)
            return 0

        lax.fori_loop(0, (KMAX_S + 1) // 2, body, 0)
        plsc.subcore_barrier()

        def wout(i, _):
            @pl.when(i % NS == sid)
            def _():
                r0 = i * 400
                pltpu.sync_copy(acc.at[pl.ds(r0, 400)],
                                out.at[cid, pl.ds(r0, 400)])
            return 0

        lax.fori_loop(0, N // 400, wout, 0)

    return scatter


_gather_h = tuple(_make_gather(h * EH) for h in range(NHALF))
_scatter_h = tuple(_make_scatter(h * EH) for h in range(NHALF))


def _prefc_body(x_ref, w_ref, b_ref, o_ref):
    o_ref[...] = jax.nn.relu(
        jnp.dot(x_ref[...], w_ref[...], preferred_element_type=jnp.float32)
        + b_ref[...]
    )


def _prefc(x, wT, b):
    bn = 2000
    return pl.pallas_call(
        _prefc_body,
        grid=(N // bn,),
        in_specs=[
            pl.BlockSpec((bn, D), lambda i: (i, 0)),
            pl.BlockSpec((D, C), lambda i: (0, 0)),
            pl.BlockSpec((1, C), lambda i: (0, 0)),
        ],
        out_specs=pl.BlockSpec((bn, C), lambda i: (i, 0)),
        out_shape=jax.ShapeDtypeStruct((N, C), jnp.float32),
    )(x, wT, b)


def _edge_body(gd_ref, gs_ref, ea_ref,
               wfd_ref, wfs_ref, wfe_ref, bf_ref,
               wsd_ref, wss_ref, wse_ref, bs_ref, msg_ref):
    gd = gd_ref[...]
    gs = gs_ref[...]
    ea = ea_ref[...]
    f = (jnp.dot(gd, wfd_ref[...], preferred_element_type=jnp.float32)
         + jnp.dot(gs, wfs_ref[...], preferred_element_type=jnp.float32)
         + jnp.dot(ea, wfe_ref[...], preferred_element_type=jnp.float32)
         + bf_ref[...])
    s = (jnp.dot(gd, wsd_ref[...], preferred_element_type=jnp.float32)
         + jnp.dot(gs, wss_ref[...], preferred_element_type=jnp.float32)
         + jnp.dot(ea, wse_ref[...], preferred_element_type=jnp.float32)
         + bs_ref[...])
    msg_ref[...] = jax.nn.sigmoid(f) * jax.nn.softplus(s)


def _edge(h, gd, gs, ea, wfd, wfs, wfe, bf, wsd, wss, wse, bs):
    be = 4000
    boff = h * (EH // be)
    espec = lambda: pl.BlockSpec((be, C), lambda i: (i + boff, 0))
    wspec = lambda shape: pl.BlockSpec(shape, lambda i: (0, 0))
    return pl.pallas_call(
        _edge_body,
        grid=(EH // be,),
        in_specs=[
            espec(),
            espec(),
            pl.BlockSpec((be, DE), lambda i: (i + boff, 0)),
            wspec((C, C)), wspec((C, C)), wspec((DE, C)), wspec((1, C)),
            wspec((C, C)), wspec((C, C)), wspec((DE, C)), wspec((1, C)),
        ],
        out_specs=pl.BlockSpec((be, C), lambda i: (i + boff, 0)),
        out_shape=jax.ShapeDtypeStruct((E, C), jnp.float32),
    )(gd, gs, ea, wfd, wfs, wfe, bf, wsd, wss, wse, bs)


def _update_body(out_ref, p0_ref, p1_ref, o_ref):
    o_ref[...] = jax.nn.relu(out_ref[...] + p0_ref[0] + p0_ref[1]
                             + p1_ref[0] + p1_ref[1])


def _update(out, part0, part1):
    bn = 2000
    pspec = pl.BlockSpec((NC, bn, C), lambda i: (0, i, 0))
    return pl.pallas_call(
        _update_body,
        grid=(N // bn,),
        in_specs=[
            pl.BlockSpec((bn, C), lambda i: (i, 0)),
            pspec,
            pspec,
        ],
        out_specs=pl.BlockSpec((bn, C), lambda i: (i, 0)),
        out_shape=jax.ShapeDtypeStruct((N, C), jnp.float32),
    )(out, part0, part1)


def _postfc_body(h_ref, wpT_ref, bp_ref, o_ref):
    o_ref[...] = jax.nn.relu(
        jnp.dot(h_ref[...], wpT_ref[...], preferred_element_type=jnp.float32)
        + bp_ref[...]
    )


def _postfc(h, wpT, bp):
    return pl.pallas_call(
        _postfc_body,
        in_specs=[
            pl.BlockSpec((N, C), lambda: (0, 0)),
            pl.BlockSpec((C, POST), lambda: (0, 0)),
            pl.BlockSpec((1, POST), lambda: (0, 0)),
        ],
        out_specs=pl.BlockSpec((N, POST), lambda: (0, 0)),
        out_shape=jax.ShapeDtypeStruct((N, POST), jnp.float32),
    )(h, wpT, bp)


def _pool_body(h_ref, batch_ref, o_ref):
    seg = batch_ref[...]
    onehot = (seg == lax.broadcasted_iota(jnp.int32, (N, G), 1)).astype(
        jnp.float32
    )
    o_ref[...] = jax.lax.dot_general(
        onehot, h_ref[...], (((0,), (0,)), ((), ())),
        preferred_element_type=jnp.float32,
        precision=lax.Precision.HIGHEST)


def _pool(h, batch2d):
    return pl.pallas_call(
        _pool_body,
        in_specs=[
            pl.BlockSpec((N, POST), lambda: (0, 0)),
            pl.BlockSpec((N, 1), lambda: (0, 0)),
        ],
        out_specs=pl.BlockSpec((G, POST), lambda: (0, 0)),
        out_shape=jax.ShapeDtypeStruct((G, POST), jnp.float32),
    )(h, batch2d)


def kernel(x, edge_index, edge_attr, batch,
           W_pre, b_pre,
           W_f0, b_f0, W_s0, b_s0,
           W_f1, b_f1, W_s1, b_s1,
           W_f2, b_f2, W_s2, b_s2,
           W_post, b_post, W_out, b_out):
    src = edge_index[0]
    dst = edge_index[1]
    zeros = jnp.zeros((N, C), jnp.float32)

    out = _prefc(x, W_pre.T, b_pre[None, :])

    for (Wf, bf, Ws, bs) in [(W_f0, b_f0, W_s0, b_s0),
                             (W_f1, b_f1, W_s1, b_s1),
                             (W_f2, b_f2, W_s2, b_s2)]:
        wfd = Wf[:, :C].T
        wfs = Wf[:, C:2 * C].T
        wfe = Wf[:, 2 * C:].T
        wsd = Ws[:, :C].T
        wss = Ws[:, C:2 * C].T
        wse = Ws[:, 2 * C:].T
        parts = []
        gh = [None] * NHALF
        msg = [None] * NHALF
        for h in range(NHALF):
            gh[h] = _gather_h[h](out, dst, src)
        for h in range(NHALF):
            msg[h] = _edge(h, gh[h][0], gh[h][1], edge_attr,
                           wfd, wfs, wfe, bf[None, :],
                           wsd, wss, wse, bs[None, :])
        for h in range(NHALF):
            parts.append(_scatter_h[h](msg[h], dst, zeros))
        out = _update(out, parts[0], parts[1])

    h = _postfc(out, W_post.T, b_post[None, :])
    seg_sum = _pool(h, batch[:, None])
    counts = jax.ops.segment_sum(jnp.ones((N,), jnp.float32), batch,
                                 num_segments=G)
    pooled = seg_sum / jnp.maximum(counts, 1.0)[:, None]
    return (pooled @ W_out.T + b_out).reshape(-1)


# speedup vs baseline: 4.1667x; 1.0031x over previous
import functools

import jax
import jax.numpy as jnp
from jax import lax
from jax.experimental import pallas as pl
from jax.experimental.pallas import tpu as pltpu
from jax.experimental.pallas import tpu_sc as plsc

N = 10000
E = 320000
D = 128
DE = 16
C = 128
POST = 64
G = 64

_SC_INFO = plsc.get_sparse_core_info()
NC = _SC_INFO.num_cores
NS = _SC_INFO.num_subcores
NW = NC * NS
NHALF = 2
EH = E // NHALF
CE = 200
NCH_G = (E // NHALF) // CE
UNITS = 2 * (NCH_G // NW)
CE_S = 160
NCH_S = EH // CE_S
KMAX_S = -(-NCH_S // NW)

_mesh = plsc.VectorSubcoreMesh(core_axis_name="c", subcore_axis_name="s")


def _make_gather(eoff):
    @functools.partial(
        pl.kernel,
        out_type=[
            jax.ShapeDtypeStruct((E, C), jnp.float32),
            jax.ShapeDtypeStruct((E, C), jnp.float32),
        ],
        mesh=_mesh,
        scratch_types=(
            [pltpu.VMEM((CE,), jnp.int32) for _ in range(4)]
            + [pltpu.VMEM((CE, C), jnp.float32) for _ in range(4)]
            + [pltpu.SemaphoreType.DMA for _ in range(12)]
        ),
    )
    def gather(table, dst, src, gd, gs, *bufs):
        idxb = bufs[0:4]
        rowsb = bufs[4:8]
        isem = bufs[8:12]
        gsem = bufs[12:16]
        wsem = bufs[16:20]
        cid = lax.axis_index("c")
        sid = lax.axis_index("s")
        wid = sid * NC + cid
        eidx = (dst, src)
        gout = (gd, gs)

        def ubase(k):
            return eoff + (wid + NW * k) * CE

        def kp(i, b):
            return 2 * i + b // 2, b % 2

        def i_start(k, p, b):
            pltpu.async_copy(eidx[p].at[pl.ds(ubase(k), CE)], idxb[b],
                             isem[b])

        def g_start(k, p, b):
            pltpu.make_async_copy(eidx[p].at[pl.ds(ubase(k), CE)],
                                  idxb[b], isem[b]).wait()
            pltpu.async_copy(table.at[idxb[b]], rowsb[b], gsem[b])

        def g_wait(b):
            pltpu.make_async_copy(table.at[idxb[b]], rowsb[b],
                                  gsem[b]).wait()

        def w_start(k, p, b):
            pltpu.async_copy(rowsb[b], gout[p].at[pl.ds(ubase(k), CE)],
                             wsem[b])

        def w_wait(k, p, b):
            pltpu.make_async_copy(rowsb[b],
                                  gout[p].at[pl.ds(ubase(k), CE)],
                                  wsem[b]).wait()

        for b in range(4):
            k, p = kp(0, b)
            i_start(k, p, b)
        for b in range(2):
            k, p = kp(0, b)
            g_start(k, p, b)

        def body(i, _):
            for b in range(4):
                b2 = (b + 2) % 4
                if b < 2:
                    k2, p2 = kp(i, b + 2)
                    km, pm = kp(i - 1, b + 2)

                    @pl.when(i >= 1)
                    def _(km=km, pm=pm, b2=b2):
                        w_wait(km, pm, b2)
                else:
                    k2, p2 = kp(i + 1, b - 2)
                    km, pm = kp(i, b - 2)
                    w_wait(km, pm, b2)
                g_start(k2, p2, b2)
                k, p = kp(i, b)
                g_wait(b)
                w_start(k, p, b)

                @pl.when(4 * i + b + 4 <= UNITS - 1)
                def _(k=k, p=p, b=b):
                    i_start(2 * (i + 1) + b // 2, b % 2, b)
            return 0

        nfull = UNITS // 4
        lax.fori_loop(0, nfull, body, 0)
        for b in range(UNITS - 4 * nfull):
            k, p = kp(nfull, b)
            g_wait(b)
            w_start(k, p, b)
        for u in range(UNITS - 4, UNITS):
            i, b = u // 4, u % 4
            k, p = kp(i, b)
            w_wait(k, p, b)

    return gather


def _make_scatter(eoff):
    @functools.partial(
        pl.kernel,
        out_type=jax.ShapeDtypeStruct((NC, N, C), jnp.float32),
        mesh=_mesh,
        scratch_types=[
            pltpu.VMEM_SHARED((N, C), jnp.float32),
            pltpu.VMEM((CE_S,), jnp.int32),
            pltpu.VMEM((CE_S,), jnp.int32),
            pltpu.VMEM((CE_S, C), jnp.float32),
            pltpu.VMEM((CE_S, C), jnp.float32),
            pltpu.SemaphoreType.DMA,
            pltpu.SemaphoreType.DMA,
            pltpu.SemaphoreType.DMA,
            pltpu.SemaphoreType.DMA,
        ],
    )
    def scatter(msg, dst, zeros, out, acc, idx0, idx1, rows0, rows1,
                msem0, msem1, isem0, isem1):
        cid = lax.axis_index("c")
        sid = lax.axis_index("s")
        wid = sid * NC + cid
        idxb = (idx0, idx1)
        rowsb = (rows0, rows1)
        msem = (msem0, msem1)
        isem = (isem0, isem1)

        @pl.when(sid == 0)
        def _():
            pltpu.sync_copy(zeros, acc)

        plsc.subcore_barrier()

        def base(k):
            return eoff + (wid + NW * k) * CE_S

        def have(k):
            return wid + NW * k < NCH_S

        def m_start(k, b):
            pltpu.async_copy(dst.at[pl.ds(base(k), CE_S)], idxb[b],
                             isem[b])
            pltpu.async_copy(msg.at[pl.ds(base(k), CE_S)], rowsb[b],
                             msem[b])

        def m_wait(k, b):
            pltpu.make_async_copy(dst.at[pl.ds(base(k), CE_S)], idxb[b],
                                  isem[b]).wait()
            pltpu.make_async_copy(msg.at[pl.ds(base(k), CE_S)], rowsb[b],
                                  msem[b]).wait()

        m_start(0, 0)

        def substep(k, b):
            @pl.when(have(k + 1))
            def _():
                m_start(k + 1, 1 - b)

            @pl.when(have(k))
            def _():
                m_wait(k, b)
                pltpu.sync_copy(rowsb[b], acc.at[idxb[b]], add=True)

        def body(i, _):
            substep(2 * i, 0)
            substep(2 * i + 1, 1)
            return 0

        lax.fori_loop(0, (KMAX_S + 1) // 2, body, 0)
        plsc.subcore_barrier()

        def wout(i, _):
            @pl.when(i % NS == sid)
            def _():
                r0 = i * 400
                pltpu.sync_copy(acc.at[pl.ds(r0, 400)],
                                out.at[cid, pl.ds(r0, 400)])
            return 0

        lax.fori_loop(0, N // 400, wout, 0)

    return scatter


_gather_h = tuple(_make_gather(h * EH) for h in range(NHALF))
_scatter_h = tuple(_make_scatter(h * EH) for h in range(NHALF))


def _prefc_body(x_ref, w_ref, b_ref, o_ref):
    o_ref[...] = jax.nn.relu(
        jnp.dot(x_ref[...], w_ref[...], preferred_element_type=jnp.float32)
        + b_ref[...]
    )


def _prefc(x, wT, b):
    bn = 2000
    return pl.pallas_call(
        _prefc_body,
        grid=(N // bn,),
        in_specs=[
            pl.BlockSpec((bn, D), lambda i: (i, 0)),
            pl.BlockSpec((D, C), lambda i: (0, 0)),
            pl.BlockSpec((1, C), lambda i: (0, 0)),
        ],
        out_specs=pl.BlockSpec((bn, C), lambda i: (i, 0)),
        out_shape=jax.ShapeDtypeStruct((N, C), jnp.float32),
    )(x, wT, b)


def _edge_body(gd_ref, gs_ref, ea_ref,
               wfd_ref, wfs_ref, wfe_ref, bf_ref,
               wsd_ref, wss_ref, wse_ref, bs_ref, msg_ref):
    gd = gd_ref[...]
    gs = gs_ref[...]
    ea = ea_ref[...]
    f = (jnp.dot(gd, wfd_ref[...], preferred_element_type=jnp.float32)
         + jnp.dot(gs, wfs_ref[...], preferred_element_type=jnp.float32)
         + jnp.dot(ea, wfe_ref[...], preferred_element_type=jnp.float32)
         + bf_ref[...])
    s = (jnp.dot(gd, wsd_ref[...], preferred_element_type=jnp.float32)
         + jnp.dot(gs, wss_ref[...], preferred_element_type=jnp.float32)
         + jnp.dot(ea, wse_ref[...], preferred_element_type=jnp.float32)
         + bs_ref[...])
    msg_ref[...] = jax.nn.sigmoid(f) * jax.nn.softplus(s)


def _edge(h, gd, gs, ea, wfd, wfs, wfe, bf, wsd, wss, wse, bs):
    be = 4000
    boff = h * (EH // be)
    espec = lambda: pl.BlockSpec((be, C), lambda i: (i + boff, 0))
    wspec = lambda shape: pl.BlockSpec(shape, lambda i: (0, 0))
    return pl.pallas_call(
        _edge_body,
        grid=(EH // be,),
        in_specs=[
            espec(),
            espec(),
            pl.BlockSpec((be, DE), lambda i: (i + boff, 0)),
            wspec((C, C)), wspec((C, C)), wspec((DE, C)), wspec((1, C)),
            wspec((C, C)), wspec((C, C)), wspec((DE, C)), wspec((1, C)),
        ],
        out_specs=pl.BlockSpec((be, C), lambda i: (i + boff, 0)),
        out_shape=jax.ShapeDtypeStruct((E, C), jnp.float32),
    )(gd, gs, ea, wfd, wfs, wfe, bf, wsd, wss, wse, bs)


def _update_body(out_ref, p0_ref, p1_ref, o_ref):
    o_ref[...] = jax.nn.relu(out_ref[...] + p0_ref[0] + p0_ref[1]
                             + p1_ref[0] + p1_ref[1])


def _update(out, part0, part1):
    bn = 2000
    pspec = pl.BlockSpec((NC, bn, C), lambda i: (0, i, 0))
    return pl.pallas_call(
        _update_body,
        grid=(N // bn,),
        in_specs=[
            pl.BlockSpec((bn, C), lambda i: (i, 0)),
            pspec,
            pspec,
        ],
        out_specs=pl.BlockSpec((bn, C), lambda i: (i, 0)),
        out_shape=jax.ShapeDtypeStruct((N, C), jnp.float32),
    )(out, part0, part1)


def _postfc_body(h_ref, wpT_ref, bp_ref, o_ref):
    o_ref[...] = jax.nn.relu(
        jnp.dot(h_ref[...], wpT_ref[...], preferred_element_type=jnp.float32)
        + bp_ref[...]
    )


def _postfc(h, wpT, bp):
    return pl.pallas_call(
        _postfc_body,
        in_specs=[
            pl.BlockSpec((N, C), lambda: (0, 0)),
            pl.BlockSpec((C, POST), lambda: (0, 0)),
            pl.BlockSpec((1, POST), lambda: (0, 0)),
        ],
        out_specs=pl.BlockSpec((N, POST), lambda: (0, 0)),
        out_shape=jax.ShapeDtypeStruct((N, POST), jnp.float32),
    )(h, wpT, bp)


def _pool_body(h_ref, batch_ref, o_ref):
    seg = batch_ref[...]
    onehot = (seg == lax.broadcasted_iota(jnp.int32, (N, G), 1)).astype(
        jnp.float32
    )
    o_ref[...] = jax.lax.dot_general(
        onehot, h_ref[...], (((0,), (0,)), ((), ())),
        preferred_element_type=jnp.float32,
        precision=lax.Precision.HIGHEST)


def _pool(h, batch2d):
    return pl.pallas_call(
        _pool_body,
        in_specs=[
            pl.BlockSpec((N, POST), lambda: (0, 0)),
            pl.BlockSpec((N, 1), lambda: (0, 0)),
        ],
        out_specs=pl.BlockSpec((G, POST), lambda: (0, 0)),
        out_shape=jax.ShapeDtypeStruct((G, POST), jnp.float32),
    )(h, batch2d)


def kernel(x, edge_index, edge_attr, batch,
           W_pre, b_pre,
           W_f0, b_f0, W_s0, b_s0,
           W_f1, b_f1, W_s1, b_s1,
           W_f2, b_f2, W_s2, b_s2,
           W_post, b_post, W_out, b_out):
    src = edge_index[0]
    dst = edge_index[1]
    zeros = jnp.zeros((N, C), jnp.float32)

    out = _prefc(x, W_pre.T, b_pre[None, :])

    for (Wf, bf, Ws, bs) in [(W_f0, b_f0, W_s0, b_s0),
                             (W_f1, b_f1, W_s1, b_s1),
                             (W_f2, b_f2, W_s2, b_s2)]:
        wfd = Wf[:, :C].T
        wfs = Wf[:, C:2 * C].T
        wfe = Wf[:, 2 * C:].T
        wsd = Ws[:, :C].T
        wss = Ws[:, C:2 * C].T
        wse = Ws[:, 2 * C:].T
        parts = []
        gh = [None] * NHALF
        msg = [None] * NHALF
        for h in range(NHALF):
            gh[h] = _gather_h[h](out, dst, src)
        for h in range(NHALF):
            msg[h] = _edge(h, gh[h][0], gh[h][1], edge_attr,
                           wfd, wfs, wfe, bf[None, :],
                           wsd, wss, wse, bs[None, :])
        for h in range(NHALF):
            parts.append(_scatter_h[h](msg[h], dst, zeros))
        out = _update(out, parts[0], parts[1])

    h = _postfc(out, W_post.T, b_post[None, :])
    seg_sum = _pool(h, batch[:, None])
    counts = jax.ops.segment_sum(jnp.ones((N,), jnp.float32), batch,
                                 num_segments=G)
    pooled = seg_sum / jnp.maximum(counts, 1.0)[:, None]
    return (pooled @ W_out.T + b_out).reshape(-1)


# speedup vs baseline: 4.2188x; 1.0125x over previous
import functools

import jax
import jax.numpy as jnp
from jax import lax
from jax.experimental import pallas as pl
from jax.experimental.pallas import tpu as pltpu
from jax.experimental.pallas import tpu_sc as plsc

N = 10000
E = 320000
D = 128
DE = 16
C = 128
POST = 64
G = 64

_SC_INFO = plsc.get_sparse_core_info()
NC = _SC_INFO.num_cores
NS = _SC_INFO.num_subcores
NW = NC * NS
NHALF = 2
EH = E // NHALF
CE = 200
NCH_G = (E // NHALF) // CE
UNITS = 2 * (NCH_G // NW)
CE_S = 160
NCH_S = EH // CE_S
KMAX_S = -(-NCH_S // NW)

_mesh = plsc.VectorSubcoreMesh(core_axis_name="c", subcore_axis_name="s")


def _make_gather(eoff):
    @functools.partial(
        pl.kernel,
        out_type=[
            jax.ShapeDtypeStruct((E, C), jnp.float32),
            jax.ShapeDtypeStruct((E, C), jnp.float32),
        ],
        mesh=_mesh,
        scratch_types=(
            [pltpu.VMEM((CE,), jnp.int32) for _ in range(4)]
            + [pltpu.VMEM((CE, C), jnp.float32) for _ in range(4)]
            + [pltpu.SemaphoreType.DMA for _ in range(12)]
        ),
    )
    def gather(table, dst, src, gd, gs, *bufs):
        idxb = bufs[0:4]
        rowsb = bufs[4:8]
        isem = bufs[8:12]
        gsem = bufs[12:16]
        wsem = bufs[16:20]
        cid = lax.axis_index("c")
        sid = lax.axis_index("s")
        wid = sid * NC + cid
        eidx = (dst, src)
        gout = (gd, gs)

        def ubase(k):
            return eoff + (wid + NW * k) * CE

        def kp(i, b):
            return 2 * i + b // 2, b % 2

        def i_start(k, p, b):
            pltpu.async_copy(eidx[p].at[pl.ds(ubase(k), CE)], idxb[b],
                             isem[b])

        def g_start(k, p, b):
            pltpu.make_async_copy(eidx[p].at[pl.ds(ubase(k), CE)],
                                  idxb[b], isem[b]).wait()
            pltpu.async_copy(table.at[idxb[b]], rowsb[b], gsem[b])

        def g_wait(b):
            pltpu.make_async_copy(table.at[idxb[b]], rowsb[b],
                                  gsem[b]).wait()

        def w_start(k, p, b):
            pltpu.async_copy(rowsb[b], gout[p].at[pl.ds(ubase(k), CE)],
                             wsem[b])

        def w_wait(k, p, b):
            pltpu.make_async_copy(rowsb[b],
                                  gout[p].at[pl.ds(ubase(k), CE)],
                                  wsem[b]).wait()

        for b in range(4):
            k, p = kp(0, b)
            i_start(k, p, b)
        for b in range(2):
            k, p = kp(0, b)
            g_start(k, p, b)

        def body(i, _):
            for b in range(4):
                b2 = (b + 2) % 4
                if b < 2:
                    k2, p2 = kp(i, b + 2)
                    km, pm = kp(i - 1, b + 2)

                    @pl.when(i >= 1)
                    def _(km=km, pm=pm, b2=b2):
                        w_wait(km, pm, b2)
                else:
                    k2, p2 = kp(i + 1, b - 2)
                    km, pm = kp(i, b - 2)
                    w_wait(km, pm, b2)
                g_start(k2, p2, b2)
                k, p = kp(i, b)
                g_wait(b)
                w_start(k, p, b)

                @pl.when(4 * i + b + 4 <= UNITS - 1)
                def _(k=k, p=p, b=b):
                    i_start(2 * (i + 1) + b // 2, b % 2, b)
            return 0

        nfull = UNITS // 4
        lax.fori_loop(0, nfull, body, 0)
        for b in range(UNITS - 4 * nfull):
            k, p = kp(nfull, b)
            g_wait(b)
            w_start(k, p, b)
        for u in range(UNITS - 4, UNITS):
            i, b = u // 4, u % 4
            k, p = kp(i, b)
            w_wait(k, p, b)

    return gather


def _make_scatter(eoff):
    @functools.partial(
        pl.kernel,
        out_type=jax.ShapeDtypeStruct((NC, N, C), jnp.float32),
        mesh=_mesh,
        scratch_types=[
            pltpu.VMEM_SHARED((N, C), jnp.float32),
            pltpu.VMEM((CE_S,), jnp.int32),
            pltpu.VMEM((CE_S,), jnp.int32),
            pltpu.VMEM((CE_S, C), jnp.float32),
            pltpu.VMEM((CE_S, C), jnp.float32),
            pltpu.SemaphoreType.DMA,
            pltpu.SemaphoreType.DMA,
            pltpu.SemaphoreType.DMA,
            pltpu.SemaphoreType.DMA,
        ],
    )
    def scatter(msg, dst, zeros, out, acc, idx0, idx1, rows0, rows1,
                msem0, msem1, isem0, isem1):
        cid = lax.axis_index("c")
        sid = lax.axis_index("s")
        wid = sid * NC + cid
        idxb = (idx0, idx1)
        rowsb = (rows0, rows1)
        msem = (msem0, msem1)
        isem = (isem0, isem1)

        @pl.when(sid == 0)
        def _():
            pltpu.sync_copy(zeros, acc)

        plsc.subcore_barrier()

        def base(k):
            return eoff + (wid + NW * k) * CE_S

        def have(k):
            return wid + NW * k < NCH_S

        def m_start(k, b):
            pltpu.async_copy(dst.at[pl.ds(base(k), CE_S)], idxb[b],
                             isem[b])
            pltpu.async_copy(msg.at[pl.ds(base(k), CE_S)], rowsb[b],
                             msem[b])

        def m_wait(k, b):
            pltpu.make_async_copy(dst.at[pl.ds(base(k), CE_S)], idxb[b],
                                  isem[b]).wait()
            pltpu.make_async_copy(msg.at[pl.ds(base(k), CE_S)], rowsb[b],
                                  msem[b]).wait()

        m_start(0, 0)

        def substep(k, b):
            @pl.when(have(k + 1))
            def _():
                m_start(k + 1, 1 - b)

            @pl.when(have(k))
            def _():
                m_wait(k, b)
                pltpu.sync_copy(rowsb[b], acc.at[idxb[b]], add=True)

        def body(i, _):
            substep(2 * i, 0)
            substep(2 * i + 1, 1)
            return 0

        lax.fori_loop(0, (KMAX_S + 1) // 2, body, 0)
        plsc.subcore_barrier()

        def wout(i, _):
            @pl.when(i % NS == sid)
            def _():
                r0 = i * 400
                pltpu.sync_copy(acc.at[pl.ds(r0, 400)],
                                out.at[cid, pl.ds(r0, 400)])
            return 0

        lax.fori_loop(0, N // 400, wout, 0)

    return scatter


_gather_h = tuple(_make_gather(h * EH) for h in range(NHALF))
_scatter_h = tuple(_make_scatter(h * EH) for h in range(NHALF))


def _prefc_body(x_ref, w_ref, b_ref, o_ref):
    o_ref[...] = jax.nn.relu(
        jnp.dot(x_ref[...], w_ref[...], preferred_element_type=jnp.float32)
        + b_ref[...]
    )


def _prefc(x, wT, b):
    bn = 2000
    return pl.pallas_call(
        _prefc_body,
        grid=(N // bn,),
        in_specs=[
            pl.BlockSpec((bn, D), lambda i: (i, 0)),
            pl.BlockSpec((D, C), lambda i: (0, 0)),
            pl.BlockSpec((1, C), lambda i: (0, 0)),
        ],
        out_specs=pl.BlockSpec((bn, C), lambda i: (i, 0)),
        out_shape=jax.ShapeDtypeStruct((N, C), jnp.float32),
    )(x, wT, b)


def _edge_body(gd_ref, gs_ref, ea_ref,
               wf_ref, bf_ref, ws_ref, bs_ref, msg_ref):
    z = jnp.concatenate([gd_ref[...], gs_ref[...], ea_ref[...]], axis=1)
    f = jnp.dot(z, wf_ref[...], preferred_element_type=jnp.float32) \
        + bf_ref[...]
    s = jnp.dot(z, ws_ref[...], preferred_element_type=jnp.float32) \
        + bs_ref[...]
    msg_ref[...] = jax.nn.sigmoid(f) * jax.nn.softplus(s)


def _edge(h, gd, gs, ea, wfT, bf, wsT, bs):
    be = 4000
    boff = h * (EH // be)
    espec = lambda: pl.BlockSpec((be, C), lambda i: (i + boff, 0))
    wspec = lambda shape: pl.BlockSpec(shape, lambda i: (0, 0))
    return pl.pallas_call(
        _edge_body,
        grid=(EH // be,),
        in_specs=[
            espec(),
            espec(),
            pl.BlockSpec((be, DE), lambda i: (i + boff, 0)),
            wspec((2 * C + DE, C)), wspec((1, C)),
            wspec((2 * C + DE, C)), wspec((1, C)),
        ],
        out_specs=pl.BlockSpec((be, C), lambda i: (i + boff, 0)),
        out_shape=jax.ShapeDtypeStruct((E, C), jnp.float32),
    )(gd, gs, ea, wfT, bf, wsT, bs)


def _update_body(out_ref, p0_ref, p1_ref, o_ref):
    o_ref[...] = jax.nn.relu(out_ref[...] + p0_ref[0] + p0_ref[1]
                             + p1_ref[0] + p1_ref[1])


def _update(out, part0, part1):
    bn = 2000
    pspec = pl.BlockSpec((NC, bn, C), lambda i: (0, i, 0))
    return pl.pallas_call(
        _update_body,
        grid=(N // bn,),
        in_specs=[
            pl.BlockSpec((bn, C), lambda i: (i, 0)),
            pspec,
            pspec,
        ],
        out_specs=pl.BlockSpec((bn, C), lambda i: (i, 0)),
        out_shape=jax.ShapeDtypeStruct((N, C), jnp.float32),
    )(out, part0, part1)


def _postfc_body(h_ref, wpT_ref, bp_ref, o_ref):
    o_ref[...] = jax.nn.relu(
        jnp.dot(h_ref[...], wpT_ref[...], preferred_element_type=jnp.float32)
        + bp_ref[...]
    )


def _postfc(h, wpT, bp):
    return pl.pallas_call(
        _postfc_body,
        in_specs=[
            pl.BlockSpec((N, C), lambda: (0, 0)),
            pl.BlockSpec((C, POST), lambda: (0, 0)),
            pl.BlockSpec((1, POST), lambda: (0, 0)),
        ],
        out_specs=pl.BlockSpec((N, POST), lambda: (0, 0)),
        out_shape=jax.ShapeDtypeStruct((N, POST), jnp.float32),
    )(h, wpT, bp)


def _pool_body(h_ref, batch_ref, o_ref):
    seg = batch_ref[...]
    onehot = (seg == lax.broadcasted_iota(jnp.int32, (N, G), 1)).astype(
        jnp.float32
    )
    o_ref[...] = jax.lax.dot_general(
        onehot, h_ref[...], (((0,), (0,)), ((), ())),
        preferred_element_type=jnp.float32,
        precision=lax.Precision.HIGHEST)


def _pool(h, batch2d):
    return pl.pallas_call(
        _pool_body,
        in_specs=[
            pl.BlockSpec((N, POST), lambda: (0, 0)),
            pl.BlockSpec((N, 1), lambda: (0, 0)),
        ],
        out_specs=pl.BlockSpec((G, POST), lambda: (0, 0)),
        out_shape=jax.ShapeDtypeStruct((G, POST), jnp.float32),
    )(h, batch2d)


def kernel(x, edge_index, edge_attr, batch,
           W_pre, b_pre,
           W_f0, b_f0, W_s0, b_s0,
           W_f1, b_f1, W_s1, b_s1,
           W_f2, b_f2, W_s2, b_s2,
           W_post, b_post, W_out, b_out):
    src = edge_index[0]
    dst = edge_index[1]
    zeros = jnp.zeros((N, C), jnp.float32)

    out = _prefc(x, W_pre.T, b_pre[None, :])

    for (Wf, bf, Ws, bs) in [(W_f0, b_f0, W_s0, b_s0),
                             (W_f1, b_f1, W_s1, b_s1),
                             (W_f2, b_f2, W_s2, b_s2)]:
        wfT = Wf.T
        wsT = Ws.T
        parts = []
        gh = [None] * NHALF
        msg = [None] * NHALF
        for h in range(NHALF):
            gh[h] = _gather_h[h](out, dst, src)
        for h in range(NHALF):
            msg[h] = _edge(h, gh[h][0], gh[h][1], edge_attr,
                           wfT, bf[None, :], wsT, bs[None, :])
        for h in range(NHALF):
            parts.append(_scatter_h[h](msg[h], dst, zeros))
        out = _update(out, parts[0], parts[1])

    h = _postfc(out, W_post.T, b_post[None, :])
    seg_sum = _pool(h, batch[:, None])
    counts = jax.ops.segment_sum(jnp.ones((N,), jnp.float32), batch,
                                 num_segments=G)
    pooled = seg_sum / jnp.maximum(counts, 1.0)[:, None]
    return (pooled @ W_out.T + b_out).reshape(-1)
